# baseline (speedup 1.0000x reference)
import functools

import numpy as np

import jax
import jax.numpy as jnp
from jax import lax
from jax.experimental import pallas as pl
from jax.experimental.pallas import tpu as pltpu
from jax.experimental.pallas import tpu_sc as plsc

B = 4096
FIELD = 26
VOCAB = 100000
EMB = 16
H = 32
NC, NS = 2, 16
NW = NC * NS
BPW = B // NW
D2 = FIELD * EMB
FPAD = VOCAB // 8 + 4

_E_NP = np.repeat(np.eye(FIELD, dtype=np.float32), EMB, axis=1)


def _repack(t2t):
    vb = VOCAB // 5
    rb = vb // 8

    def body(in_ref, out_ref):
        for c in range(5):
            x = in_ref[0, :, pl.ds(c * vb, vb)]
            x3 = jnp.reshape(x.T, (rb, 8, EMB))
            y = jnp.concatenate([x3[:, j, :] for j in range(8)], axis=1)
            out_ref[pl.ds(c * rb, rb), :] = y

    return pl.pallas_call(
        body,
        grid=(FIELD,),
        in_specs=[pl.BlockSpec((1, EMB, VOCAB), lambda i: (i, 0, 0))],
        out_specs=pl.BlockSpec((FPAD, 128), lambda i: (i, 0)),
        out_shape=jax.ShapeDtypeStruct((FIELD * FPAD, 128), jnp.float32),
    )(t2t)


def _sc_gather(idx128, poff, idx1, t2p, t1):
    mesh = plsc.VectorSubcoreMesh(core_axis_name="c", subcore_axis_name="s")

    @functools.partial(
        pl.kernel,
        out_type=(
            jax.ShapeDtypeStruct((B, D2), jnp.float32),
            jax.ShapeDtypeStruct((FIELD, B), jnp.float32),
        ),
        mesh=mesh,
        scratch_types=[
            pltpu.VMEM((FIELD, BPW), jnp.int32),
            pltpu.VMEM((FIELD, BPW), jnp.int32),
            pltpu.VMEM((FIELD, BPW), jnp.int32),
            pltpu.VMEM((2, BPW, 128), jnp.float32),
            pltpu.VMEM((BPW, D2), jnp.float32),
            pltpu.VMEM((FIELD, BPW), jnp.float32),
            pltpu.SemaphoreType.DMA,
            pltpu.SemaphoreType.DMA,
            pltpu.SemaphoreType.DMA,
        ],
        compiler_params=pltpu.CompilerParams(needs_layout_passes=False),
    )
    def k(idx128_hbm, poff_hbm, idx1_hbm, t2_hbm, t1_hbm, out2_hbm, out1_hbm,
          idx128_v, poff_v, idx1_v, buf_v, outv, rows1_v, sem_a, sem_b, sem1):
        wid = lax.axis_index("s") * NC + lax.axis_index("c")
        b0 = wid * BPW
        pltpu.sync_copy(idx128_hbm.at[wid], idx128_v)
        pltpu.sync_copy(poff_hbm.at[wid], poff_v)
        pltpu.sync_copy(idx1_hbm.at[wid], idx1_v)

        def fire1(f, carry):
            pltpu.async_copy(t1_hbm.at[idx1_v.at[f]], rows1_v.at[f], sem1)
            return carry
        lax.fori_loop(0, FIELD, fire1, 0)

        pltpu.async_copy(t2_hbm.at[idx128_v.at[0]], buf_v.at[0], sem_a)

        iota16 = lax.iota(jnp.int32, 16)
        zeros16 = jnp.zeros((16,), jnp.int32)

        def select_field(f, slot):
            bslot = buf_v.at[slot]
            pref = poff_v.at[f]
            cbase = zeros16 + f * EMB
            for g in range(BPW // 16):
                rows = iota16 + (g * 16)
                prow = pref[pl.ds(g * 16, 16)]
                for e in range(EMB):
                    val = plsc.load_gather(bslot, [rows, prow + e])
                    plsc.store_scatter(outv, [rows, cbase + e], val)

        def drain_slot(slot, sem):
            pltpu.make_async_copy(t2_hbm.at[pl.ds(0, BPW)], buf_v.at[slot],
                                  sem).wait()

        def body(t, carry):
            fe = 2 * t
            pltpu.async_copy(t2_hbm.at[idx128_v.at[fe + 1]], buf_v.at[1],
                             sem_b)
            drain_slot(0, sem_a)
            select_field(fe, 0)

            @pl.when(fe + 2 < FIELD)
            def _():
                pltpu.async_copy(t2_hbm.at[idx128_v.at[fe + 2]], buf_v.at[0],
                                 sem_a)

            drain_slot(1, sem_b)
            select_field(fe + 1, 1)
            return carry

        lax.fori_loop(0, FIELD // 2, body, 0)

        pltpu.sync_copy(outv, out2_hbm.at[pl.ds(b0, BPW)])
        pltpu.make_async_copy(out1_hbm.at[:, pl.ds(0, BPW)], rows1_v,
                              sem1).wait()
        pltpu.sync_copy(rows1_v, out1_hbm.at[:, pl.ds(b0, BPW)])

    return k(idx128, poff, idx1, t2p, t1)


def _mlp(g1, g2, xv, e_mat, w1f, w1s, c1, w2, b2, w3, b3):
    blk = 512

    def body(g1_ref, g2_ref, xv_ref, e_ref, w1f_ref, w1s_ref, c1_ref,
             w2_ref, b2_ref, w3_ref, b3_ref, out_ref):
        xv_b = xv_ref[...]
        ff = g1_ref[...] * xv_b
        xv16 = jnp.dot(xv_b, e_ref[...], preferred_element_type=jnp.float32)
        fs = g2_ref[...] * xv16
        h = jnp.tanh(
            jnp.dot(ff, w1f_ref[...], preferred_element_type=jnp.float32)
            + jnp.dot(fs, w1s_ref[...], preferred_element_type=jnp.float32)
            + c1_ref[...])
        h = jnp.tanh(
            jnp.dot(h, w2_ref[...], preferred_element_type=jnp.float32)
            + b2_ref[...])
        out_ref[...] = (
            jnp.dot(h, w3_ref[...], preferred_element_type=jnp.float32)
            + b3_ref[...])

    out = pl.pallas_call(
        body,
        grid=(B // blk,),
        in_specs=[
            pl.BlockSpec((blk, FIELD), lambda i: (i, 0)),
            pl.BlockSpec((blk, D2), lambda i: (i, 0)),
            pl.BlockSpec((blk, FIELD), lambda i: (i, 0)),
            pl.BlockSpec((FIELD, D2), lambda i: (0, 0)),
            pl.BlockSpec((FIELD, H), lambda i: (0, 0)),
            pl.BlockSpec((D2, H), lambda i: (0, 0)),
            pl.BlockSpec((1, H), lambda i: (0, 0)),
            pl.BlockSpec((H, H), lambda i: (0, 0)),
            pl.BlockSpec((1, H), lambda i: (0, 0)),
            pl.BlockSpec((H, 1), lambda i: (0, 0)),
            pl.BlockSpec((1, 1), lambda i: (0, 0)),
        ],
        out_specs=pl.BlockSpec((blk, 1), lambda i: (i, 0)),
        out_shape=jax.ShapeDtypeStruct((B, 1), jnp.float32),
    )(g1, g2, xv, e_mat, w1f, w1s, c1, w2, b2, w3, b3)
    return out[:, 0]


def kernel(Xi, Xv, fm_bias, first_tables, second_tables, W1, b1, W2, b2, W3, b3):
    idx_t = Xi[:, :, 0].T
    idxw = idx_t.reshape(FIELD, NW, BPW).transpose(1, 0, 2)
    offs = (jnp.arange(FIELD, dtype=jnp.int32) * VOCAB)[None, :, None]
    offs_pad = (jnp.arange(FIELD, dtype=jnp.int32) * FPAD)[None, :, None]
    flat = idxw + offs
    idx128 = idxw // 8 + offs_pad
    poff = (idxw % 8) * EMB
    t2t = second_tables.transpose(0, 2, 1)
    t2p = _repack(t2t)
    t1 = first_tables.reshape(FIELD * VOCAB)
    g2, g1t = _sc_gather(idx128, poff, flat, t2p, t1)
    e_mat = jnp.asarray(_E_NP)
    c1 = (fm_bias[0] * W1[0] + b1)[None, :]
    return _mlp(g1t.T, g2, Xv, e_mat, W1[1:1 + FIELD], W1[1 + FIELD:], c1,
                W2, b2[None, :], W3, b3[None, :])


# speedup vs baseline: 1.3477x; 1.3477x over previous
import functools

import numpy as np

import jax
import jax.numpy as jnp
from jax import lax
from jax.experimental import pallas as pl
from jax.experimental.pallas import tpu as pltpu
from jax.experimental.pallas import tpu_sc as plsc

B = 4096
FIELD = 26
VOCAB = 100000
EMB = 16
H = 32
N = B * FIELD
NC, NS = 2, 16
NW = NC * NS
PER_W = N // NW
CHUNK = 128
NCH = PER_W // CHUNK

_E_NP = np.repeat(np.eye(FIELD, dtype=np.float32), EMB, axis=1)


def _sc_gather(flat_idx, t2, t1):
    mesh = plsc.VectorSubcoreMesh(core_axis_name="c", subcore_axis_name="s")

    @functools.partial(
        pl.kernel,
        out_type=(
            jax.ShapeDtypeStruct((N, EMB), jnp.float32),
            jax.ShapeDtypeStruct((N,), jnp.float32),
        ),
        mesh=mesh,
        scratch_types=[
            pltpu.VMEM((NCH, CHUNK), jnp.int32),
            pltpu.VMEM((PER_W, EMB), jnp.float32),
            pltpu.VMEM((PER_W,), jnp.float32),
            pltpu.SemaphoreType.DMA,
            pltpu.SemaphoreType.DMA,
        ],
        compiler_params=pltpu.CompilerParams(use_tc_tiling_on_sc=False),
    )
    def k(idx_hbm, t2_hbm, t1_hbm, out2_hbm, out1_hbm,
          idx_v, rows2_v, rows1_v, sem2, sem1):
        wid = lax.axis_index("s") * NC + lax.axis_index("c")
        base = wid * PER_W
        pltpu.sync_copy(idx_hbm.at[wid], idx_v)

        def fire(j, carry):
            pltpu.async_copy(
                t2_hbm.at[idx_v.at[j]], rows2_v.at[pl.ds(j * CHUNK, CHUNK)],
                sem2)
            pltpu.async_copy(
                t1_hbm.at[idx_v.at[j]], rows1_v.at[pl.ds(j * CHUNK, CHUNK)],
                sem1)
            return carry

        lax.fori_loop(0, NCH, fire, 0)
        pltpu.make_async_copy(t2_hbm.at[pl.ds(0, PER_W)], rows2_v, sem2).wait()
        pltpu.make_async_copy(t1_hbm.at[pl.ds(0, PER_W)], rows1_v, sem1).wait()
        pltpu.sync_copy(rows2_v, out2_hbm.at[pl.ds(base, PER_W)])
        pltpu.sync_copy(rows1_v, out1_hbm.at[pl.ds(base, PER_W)])

    return k(flat_idx, t2, t1)


def _mlp(g1, g2, xv, e_mat, w1f, w1s, c1, w2, b2, w3, b3):
    blk = 512
    d2 = FIELD * EMB

    def body(g1_ref, g2_ref, xv_ref, e_ref, w1f_ref, w1s_ref, c1_ref,
             w2_ref, b2_ref, w3_ref, b3_ref, out_ref):
        xv_b = xv_ref[...]
        ff = g1_ref[...] * xv_b
        xv16 = jnp.dot(xv_b, e_ref[...], preferred_element_type=jnp.float32)
        fs = g2_ref[...] * xv16
        h = jnp.tanh(
            jnp.dot(ff, w1f_ref[...], preferred_element_type=jnp.float32)
            + jnp.dot(fs, w1s_ref[...], preferred_element_type=jnp.float32)
            + c1_ref[...])
        h = jnp.tanh(
            jnp.dot(h, w2_ref[...], preferred_element_type=jnp.float32)
            + b2_ref[...])
        out_ref[...] = (
            jnp.dot(h, w3_ref[...], preferred_element_type=jnp.float32)
            + b3_ref[...])

    out = pl.pallas_call(
        body,
        grid=(B // blk,),
        in_specs=[
            pl.BlockSpec((blk, FIELD), lambda i: (i, 0)),
            pl.BlockSpec((blk, d2), lambda i: (i, 0)),
            pl.BlockSpec((blk, FIELD), lambda i: (i, 0)),
            pl.BlockSpec((FIELD, d2), lambda i: (0, 0)),
            pl.BlockSpec((FIELD, H), lambda i: (0, 0)),
            pl.BlockSpec((d2, H), lambda i: (0, 0)),
            pl.BlockSpec((1, H), lambda i: (0, 0)),
            pl.BlockSpec((H, H), lambda i: (0, 0)),
            pl.BlockSpec((1, H), lambda i: (0, 0)),
            pl.BlockSpec((H, 1), lambda i: (0, 0)),
            pl.BlockSpec((1, 1), lambda i: (0, 0)),
        ],
        out_specs=pl.BlockSpec((blk, 1), lambda i: (i, 0)),
        out_shape=jax.ShapeDtypeStruct((B, 1), jnp.float32),
    )(g1, g2, xv, e_mat, w1f, w1s, c1, w2, b2, w3, b3)
    return out[:, 0]


def kernel(Xi, Xv, fm_bias, first_tables, second_tables, W1, b1, W2, b2, W3, b3):
    idx = Xi[:, :, 0]
    offs = (jnp.arange(FIELD, dtype=jnp.int32) * VOCAB)[None, :]
    flat_idx = (idx + offs).reshape(NW, NCH, CHUNK)
    t2 = second_tables.reshape(FIELD * VOCAB, EMB)
    t1 = first_tables.reshape(FIELD * VOCAB)
    rows2, rows1 = _sc_gather(flat_idx, t2, t1)
    g2 = rows2.reshape(B, FIELD * EMB)
    g1 = rows1.reshape(B, FIELD)
    e_mat = jnp.asarray(_E_NP)
    c1 = (fm_bias[0] * W1[0] + b1)[None, :]
    return _mlp(g1, g2, Xv, e_mat, W1[1:1 + FIELD], W1[1 + FIELD:], c1,
                W2, b2[None, :], W3, b3[None, :])
